# Initial kernel scaffold; baseline (speedup 1.0000x reference)
import jax
import jax.numpy as jnp
from jax.experimental import pallas as pl


def kernel(x, edge_index, W1, b1, gamma1, beta1, W2, b2, gamma2, beta2):
    raise NotImplementedError("write your pallas kernel here")


# speedup vs baseline: 5.0015x; 5.0015x over previous
import functools

import jax
import jax.numpy as jnp
from jax import lax
from jax.experimental import pallas as pl
from jax.experimental.pallas import tpu as pltpu
from jax.experimental.pallas import tpu_sc as plsc

N = 10000
E = 320000
D = 128

NC = 2
NS = 16
NW = NC * NS
FPT = D // NW
EPW = E // NW
EC = 8000
L = 16

@functools.cache
def _mesh():
    return plsc.VectorSubcoreMesh(core_axis_name="c", subcore_axis_name="s",
                                  num_cores=NC, num_subcores=NS)


def _wid():
    return lax.axis_index("s") * NC + lax.axis_index("c")


def _zero_fill(ref, nwords):
    z = jnp.zeros((L,), jnp.float32)

    def body(i, _):
        ref[pl.ds(i * L, L)] = z
        return 0

    lax.fori_loop(0, nwords // L, body, 0)


@functools.cache
def _hist_sc_kernel():
    return pl.kernel(
        _hist_sc_body,
        out_type=jax.ShapeDtypeStruct((NW, N), jnp.float32),
        mesh=_mesh(),
        scratch_types=[
            pltpu.VMEM((EPW,), jnp.int32),
            pltpu.VMEM((N,), jnp.float32),
        ],
        compiler_params=pltpu.CompilerParams(needs_layout_passes=False),
    )


def _hist_sc_body(col_hbm, out_hbm, col_v, hist_v):
    w = _wid()
    pltpu.sync_copy(col_hbm.at[pl.ds(w * EPW, EPW)], col_v)
    _zero_fill(hist_v, N)
    ones = jnp.ones((L,), jnp.float32)

    def body(j, _):
        cols = col_v[pl.ds(j * L, L)]
        plsc.addupdate_scatter(hist_v, [cols], ones)
        return 0

    lax.fori_loop(0, EPW // L, body, 0)
    pltpu.sync_copy(hist_v, out_hbm.at[w])


@functools.cache
def _agg_sc_kernel():
    return pl.kernel(
        _agg_sc_body,
        out_type=jax.ShapeDtypeStruct((NW, N * FPT), jnp.float32),
        mesh=_mesh(),
        scratch_types=[
            pltpu.VMEM((N * FPT,), jnp.float32),
            pltpu.VMEM((N * FPT,), jnp.float32),
            pltpu.VMEM((N,), jnp.float32),
            pltpu.VMEM((EC,), jnp.int32),
            pltpu.VMEM((EC,), jnp.int32),
        ],
        compiler_params=pltpu.CompilerParams(needs_layout_passes=False),
    )


def _agg_sc_body(row_hbm, col_hbm, dis_hbm, lin_hbm, out_hbm,
                 lin_v, acc_v, dis_v, row_v, col_v):
    w = _wid()
    pltpu.sync_copy(lin_hbm.at[w], lin_v)
    pltpu.sync_copy(dis_hbm, dis_v)
    _zero_fill(acc_v, N * FPT)

    def chunk(c, _):
        pltpu.sync_copy(row_hbm.at[pl.ds(c * EC, EC)], row_v)
        pltpu.sync_copy(col_hbm.at[pl.ds(c * EC, EC)], col_v)

        def grp(j, _):
            rows = row_v[pl.ds(j * L, L)]
            cols = col_v[pl.ds(j * L, L)]
            nrm = plsc.load_gather(dis_v, [rows]) * plsc.load_gather(dis_v, [cols])
            rb = rows * FPT
            cb = cols * FPT
            for f in range(FPT):
                v = plsc.load_gather(lin_v, [rb + f])
                plsc.addupdate_scatter(acc_v, [cb + f], v * nrm)
            return 0

        lax.fori_loop(0, EC // L, grp, 0)
        return 0

    lax.fori_loop(0, E // EC, chunk, 0)
    pltpu.sync_copy(acc_v, out_hbm.at[w])


def _deg_body(parts_ref, dis_ref, inv_ref):
    deg = jnp.sum(parts_ref[...], axis=0) + 1.0
    inv = 1.0 / deg
    inv_ref[...] = inv
    dis_ref[...] = jnp.sqrt(inv)


def _deg_finish(parts):
    return pl.pallas_call(
        _deg_body,
        out_shape=(
            jax.ShapeDtypeStruct((N,), jnp.float32),
            jax.ShapeDtypeStruct((N,), jnp.float32),
        ),
    )(parts)


def _mm_body(x_ref, w_ref, o_ref):
    o_ref[...] = jnp.dot(x_ref[...], w_ref[...],
                         preferred_element_type=jnp.float32)


def _matmul(x, w):
    return pl.pallas_call(
        _mm_body,
        out_shape=jax.ShapeDtypeStruct((N, D), jnp.float32),
    )(x, w)


def _bn_relu(t, gamma, beta):
    m = jnp.mean(t, axis=0)
    v = jnp.mean(t * t, axis=0) - m * m
    h = (t - m) * lax.rsqrt(v + 1e-5) * gamma + beta
    return jnp.maximum(h, 0.0)


def _mid_body(agg_ref, lin_ref, inv_ref, b_ref, g_ref, be_ref, w2_ref, o_ref):
    inv = inv_ref[...].reshape(N, 1)
    t = agg_ref[...] + inv * lin_ref[...] + b_ref[...]
    h = _bn_relu(t, g_ref[...], be_ref[...])
    o_ref[...] = jnp.dot(h, w2_ref[...], preferred_element_type=jnp.float32)


def _layer_mid(agg, lin, inv_deg, b, gamma, beta, w2):
    return pl.pallas_call(
        _mid_body,
        out_shape=jax.ShapeDtypeStruct((N, D), jnp.float32),
    )(agg, lin, inv_deg, b, gamma, beta, w2)


def _out_body(agg_ref, lin_ref, inv_ref, b_ref, g_ref, be_ref, o_ref):
    inv = inv_ref[...].reshape(N, 1)
    t = agg_ref[...] + inv * lin_ref[...] + b_ref[...]
    o_ref[...] = _bn_relu(t, g_ref[...], be_ref[...])


def _layer_out(agg, lin, inv_deg, b, gamma, beta):
    return pl.pallas_call(
        _out_body,
        out_shape=jax.ShapeDtypeStruct((N, D), jnp.float32),
    )(agg, lin, inv_deg, b, gamma, beta)


def _to_sc(lin):
    return lin.reshape(N, NW, FPT).transpose(1, 0, 2).reshape(NW, N * FPT)


def _from_sc(agg):
    return agg.reshape(NW, N, FPT).transpose(1, 0, 2).reshape(N, D)


def kernel(x, edge_index, W1, b1, gamma1, beta1, W2, b2, gamma2, beta2):
    row = edge_index[0].astype(jnp.int32)
    col = edge_index[1].astype(jnp.int32)

    parts = _hist_sc_kernel()(col)
    dis, inv_deg = _deg_finish(parts)

    lin1 = _matmul(x, W1)
    agg1 = _from_sc(_agg_sc_kernel()(row, col, dis, _to_sc(lin1)))
    lin2 = _layer_mid(agg1, lin1, inv_deg, b1, gamma1, beta1, W2)
    agg2 = _from_sc(_agg_sc_kernel()(row, col, dis, _to_sc(lin2)))
    return _layer_out(agg2, lin2, inv_deg, b2, gamma2, beta2)


# speedup vs baseline: 5.6256x; 1.1248x over previous
import functools

import jax
import jax.numpy as jnp
from jax import lax
from jax.experimental import pallas as pl
from jax.experimental.pallas import tpu as pltpu
from jax.experimental.pallas import tpu_sc as plsc

N = 10000
E = 320000
D = 128

NC = 2
NS = 16
NW = NC * NS
FPT = D // NW
EPW = E // NW
EC = 8000
L = 16
UNROLL = 4


@functools.cache
def _mesh():
    return plsc.VectorSubcoreMesh(core_axis_name="c", subcore_axis_name="s",
                                  num_cores=NC, num_subcores=NS)


def _wid():
    return lax.axis_index("s") * NC + lax.axis_index("c")


def _zero_fill(ref, nwords):
    z = jnp.zeros((L,), jnp.float32)

    def body(i, _):
        ref[pl.ds(i * L, L)] = z
        return 0

    lax.fori_loop(0, nwords // L, body, 0)


@functools.cache
def _hist_sc_kernel():
    return pl.kernel(
        _hist_sc_body,
        out_type=jax.ShapeDtypeStruct((NW, N), jnp.float32),
        mesh=_mesh(),
        scratch_types=[
            pltpu.VMEM((EPW,), jnp.int32),
            pltpu.VMEM((N,), jnp.float32),
        ],
        compiler_params=pltpu.CompilerParams(needs_layout_passes=False),
    )


def _hist_sc_body(col_hbm, out_hbm, col_v, hist_v):
    w = _wid()
    pltpu.sync_copy(col_hbm.at[pl.ds(w * EPW, EPW)], col_v)
    _zero_fill(hist_v, N)
    ones = jnp.ones((L,), jnp.float32)

    def body(j, _):
        cols = col_v[pl.ds(j * L, L)]
        plsc.addupdate_scatter(hist_v, [cols], ones)
        return 0

    lax.fori_loop(0, EPW // L, body, 0)
    pltpu.sync_copy(hist_v, out_hbm.at[w])


@functools.cache
def _agg_sc_kernel():
    return pl.kernel(
        _agg_sc_body,
        out_type=jax.ShapeDtypeStruct((NW, N * FPT), jnp.float32),
        mesh=_mesh(),
        scratch_types=[
            pltpu.VMEM((N * FPT,), jnp.float32),
            pltpu.VMEM((N * FPT,), jnp.float32),
            pltpu.VMEM((EC,), jnp.int32),
            pltpu.VMEM((EC,), jnp.int32),
        ],
        compiler_params=pltpu.CompilerParams(needs_layout_passes=False),
    )


def _agg_sc_body(row_hbm, col_hbm, lin_hbm, out_hbm,
                 lin_v, acc_v, row_v, col_v):
    w = _wid()
    pltpu.sync_copy(lin_hbm.at[w], lin_v)
    _zero_fill(acc_v, N * FPT)

    def chunk(c, _):
        pltpu.sync_copy(row_hbm.at[pl.ds(c * EC, EC)], row_v)
        pltpu.sync_copy(col_hbm.at[pl.ds(c * EC, EC)], col_v)

        def grp(j, _):
            for u in range(UNROLL):
                o = (j * UNROLL + u) * L
                rows = row_v[pl.ds(o, L)]
                cols = col_v[pl.ds(o, L)]
                rb = rows * FPT
                cb = cols * FPT
                for f in range(FPT):
                    v = plsc.load_gather(lin_v, [rb + f])
                    plsc.addupdate_scatter(acc_v, [cb + f], v)
            return 0

        lax.fori_loop(0, EC // (L * UNROLL), grp, 0)
        return 0

    lax.fori_loop(0, E // EC, chunk, 0)
    pltpu.sync_copy(acc_v, out_hbm.at[w])


def _deg_body(parts_ref, dis_ref, inv_ref):
    deg = jnp.sum(parts_ref[...], axis=0) + 1.0
    inv = 1.0 / deg
    inv_ref[...] = inv
    dis_ref[...] = jnp.sqrt(inv)


def _deg_finish(parts):
    return pl.pallas_call(
        _deg_body,
        out_shape=(
            jax.ShapeDtypeStruct((N,), jnp.float32),
            jax.ShapeDtypeStruct((N,), jnp.float32),
        ),
    )(parts)


def _mm_body(x_ref, w_ref, dis_ref, lin_ref, lins_ref):
    lin = jnp.dot(x_ref[...], w_ref[...], preferred_element_type=jnp.float32)
    lin_ref[...] = lin
    lins_ref[...] = lin * dis_ref[...].reshape(N, 1)


def _matmul(x, w, dis):
    return pl.pallas_call(
        _mm_body,
        out_shape=(
            jax.ShapeDtypeStruct((N, D), jnp.float32),
            jax.ShapeDtypeStruct((N, D), jnp.float32),
        ),
    )(x, w, dis)


def _bn_relu(t, gamma, beta):
    m = jnp.mean(t, axis=0)
    v = jnp.mean(t * t, axis=0) - m * m
    h = (t - m) * lax.rsqrt(v + 1e-5) * gamma + beta
    return jnp.maximum(h, 0.0)


def _mid_body(acc_ref, lin_ref, dis_ref, inv_ref, b_ref, g_ref, be_ref,
              w2_ref, lin2_ref, lin2s_ref):
    dis = dis_ref[...].reshape(N, 1)
    inv = inv_ref[...].reshape(N, 1)
    t = dis * acc_ref[...] + inv * lin_ref[...] + b_ref[...]
    h = _bn_relu(t, g_ref[...], be_ref[...])
    lin2 = jnp.dot(h, w2_ref[...], preferred_element_type=jnp.float32)
    lin2_ref[...] = lin2
    lin2s_ref[...] = lin2 * dis


def _layer_mid(acc, lin, dis, inv_deg, b, gamma, beta, w2):
    return pl.pallas_call(
        _mid_body,
        out_shape=(
            jax.ShapeDtypeStruct((N, D), jnp.float32),
            jax.ShapeDtypeStruct((N, D), jnp.float32),
        ),
    )(acc, lin, dis, inv_deg, b, gamma, beta, w2)


def _out_body(acc_ref, lin_ref, dis_ref, inv_ref, b_ref, g_ref, be_ref, o_ref):
    dis = dis_ref[...].reshape(N, 1)
    inv = inv_ref[...].reshape(N, 1)
    t = dis * acc_ref[...] + inv * lin_ref[...] + b_ref[...]
    o_ref[...] = _bn_relu(t, g_ref[...], be_ref[...])


def _layer_out(acc, lin, dis, inv_deg, b, gamma, beta):
    return pl.pallas_call(
        _out_body,
        out_shape=jax.ShapeDtypeStruct((N, D), jnp.float32),
    )(acc, lin, dis, inv_deg, b, gamma, beta)


def _to_sc(lin):
    return lin.reshape(N, NW, FPT).transpose(1, 0, 2).reshape(NW, N * FPT)


def _from_sc(acc):
    return acc.reshape(NW, N, FPT).transpose(1, 0, 2).reshape(N, D)


def kernel(x, edge_index, W1, b1, gamma1, beta1, W2, b2, gamma2, beta2):
    row = edge_index[0].astype(jnp.int32)
    col = edge_index[1].astype(jnp.int32)

    parts = _hist_sc_kernel()(col)
    dis, inv_deg = _deg_finish(parts)

    lin1, lin1s = _matmul(x, W1, dis)
    acc1 = _from_sc(_agg_sc_kernel()(row, col, _to_sc(lin1s)))
    lin2, lin2s = _layer_mid(acc1, lin1, dis, inv_deg, b1, gamma1, beta1, W2)
    acc2 = _from_sc(_agg_sc_kernel()(row, col, _to_sc(lin2s)))
    return _layer_out(acc2, lin2, dis, inv_deg, b2, gamma2, beta2)


# speedup vs baseline: 10.1074x; 1.7967x over previous
import functools

import jax
import jax.numpy as jnp
from jax import lax
from jax.experimental import pallas as pl
from jax.experimental.pallas import tpu as pltpu
from jax.experimental.pallas import tpu_sc as plsc

N = 10000
E = 320000
D = 128

NC = 2
NS = 16
NW = NC * NS
FPT = D // NW
EPW = E // NW
EC = 8000
L = 16
UNROLL = 4


@functools.cache
def _mesh():
    return plsc.VectorSubcoreMesh(core_axis_name="c", subcore_axis_name="s",
                                  num_cores=NC, num_subcores=NS)


def _wid():
    return lax.axis_index("s") * NC + lax.axis_index("c")


def _zero_fill(ref, nwords):
    z = jnp.zeros((L,), jnp.float32)

    def body(i, _):
        ref[pl.ds(i * L, L)] = z
        return 0

    lax.fori_loop(0, nwords // L, body, 0)


@functools.cache
def _hist_sc_kernel():
    return pl.kernel(
        _hist_sc_body,
        out_type=jax.ShapeDtypeStruct((NW, N), jnp.float32),
        mesh=_mesh(),
        scratch_types=[
            pltpu.VMEM((EPW,), jnp.int32),
            pltpu.VMEM((N,), jnp.float32),
        ],
        compiler_params=pltpu.CompilerParams(needs_layout_passes=False),
    )


def _hist_sc_body(col_hbm, out_hbm, col_v, hist_v):
    w = _wid()
    pltpu.sync_copy(col_hbm.at[pl.ds(w * EPW, EPW)], col_v)
    _zero_fill(hist_v, N)
    ones = jnp.ones((L,), jnp.float32)

    def body(j, _):
        cols = col_v[pl.ds(j * L, L)]
        plsc.addupdate_scatter(hist_v, [cols], ones)
        return 0

    lax.fori_loop(0, EPW // L, body, 0)
    pltpu.sync_copy(hist_v, out_hbm.at[w])


@functools.cache
def _agg_sc_kernel():
    return pl.kernel(
        _agg_sc_body,
        out_type=jax.ShapeDtypeStruct((NW, N * FPT), jnp.float32),
        mesh=_mesh(),
        scratch_types=[
            pltpu.VMEM((N * FPT,), jnp.float32),
            pltpu.VMEM((N * FPT,), jnp.float32),
            pltpu.VMEM((EC,), jnp.int32),
            pltpu.VMEM((EC,), jnp.int32),
        ],
        compiler_params=pltpu.CompilerParams(needs_layout_passes=False),
    )


def _agg_sc_body(row_hbm, col_hbm, lin_hbm, out_hbm,
                 lin_v, acc_v, row_v, col_v):
    w = _wid()
    pltpu.sync_copy(lin_hbm.at[w], lin_v)
    _zero_fill(acc_v, N * FPT)

    def chunk(c, _):
        pltpu.sync_copy(row_hbm.at[pl.ds(c * EC, EC)], row_v)
        pltpu.sync_copy(col_hbm.at[pl.ds(c * EC, EC)], col_v)

        @plsc.parallel_loop(0, EC // L, unroll=UNROLL)
        def grp(j):
            o = j * L
            rows = row_v[pl.ds(o, L)]
            cols = col_v[pl.ds(o, L)]
            rb = rows * FPT
            cb = cols * FPT
            for f in range(FPT):
                v = plsc.load_gather(lin_v, [rb + f])
                plsc.addupdate_scatter(acc_v, [cb + f], v)

        return 0

    lax.fori_loop(0, E // EC, chunk, 0)
    pltpu.sync_copy(acc_v, out_hbm.at[w])


def _deg_body(parts_ref, dis_ref, inv_ref):
    deg = jnp.sum(parts_ref[...], axis=0) + 1.0
    inv = 1.0 / deg
    inv_ref[...] = inv
    dis_ref[...] = jnp.sqrt(inv)


def _deg_finish(parts):
    return pl.pallas_call(
        _deg_body,
        out_shape=(
            jax.ShapeDtypeStruct((N,), jnp.float32),
            jax.ShapeDtypeStruct((N,), jnp.float32),
        ),
    )(parts)


def _mm_body(x_ref, w_ref, dis_ref, lin_ref, lins_ref):
    lin = jnp.dot(x_ref[...], w_ref[...], preferred_element_type=jnp.float32)
    lin_ref[...] = lin
    lins_ref[...] = lin * dis_ref[...].reshape(N, 1)


def _matmul(x, w, dis):
    return pl.pallas_call(
        _mm_body,
        out_shape=(
            jax.ShapeDtypeStruct((N, D), jnp.float32),
            jax.ShapeDtypeStruct((N, D), jnp.float32),
        ),
    )(x, w, dis)


def _bn_relu(t, gamma, beta):
    m = jnp.mean(t, axis=0)
    v = jnp.mean(t * t, axis=0) - m * m
    h = (t - m) * lax.rsqrt(v + 1e-5) * gamma + beta
    return jnp.maximum(h, 0.0)


def _mid_body(acc_ref, lin_ref, dis_ref, inv_ref, b_ref, g_ref, be_ref,
              w2_ref, lin2_ref, lin2s_ref):
    dis = dis_ref[...].reshape(N, 1)
    inv = inv_ref[...].reshape(N, 1)
    t = dis * acc_ref[...] + inv * lin_ref[...] + b_ref[...]
    h = _bn_relu(t, g_ref[...], be_ref[...])
    lin2 = jnp.dot(h, w2_ref[...], preferred_element_type=jnp.float32)
    lin2_ref[...] = lin2
    lin2s_ref[...] = lin2 * dis


def _layer_mid(acc, lin, dis, inv_deg, b, gamma, beta, w2):
    return pl.pallas_call(
        _mid_body,
        out_shape=(
            jax.ShapeDtypeStruct((N, D), jnp.float32),
            jax.ShapeDtypeStruct((N, D), jnp.float32),
        ),
    )(acc, lin, dis, inv_deg, b, gamma, beta, w2)


def _out_body(acc_ref, lin_ref, dis_ref, inv_ref, b_ref, g_ref, be_ref, o_ref):
    dis = dis_ref[...].reshape(N, 1)
    inv = inv_ref[...].reshape(N, 1)
    t = dis * acc_ref[...] + inv * lin_ref[...] + b_ref[...]
    o_ref[...] = _bn_relu(t, g_ref[...], be_ref[...])


def _layer_out(acc, lin, dis, inv_deg, b, gamma, beta):
    return pl.pallas_call(
        _out_body,
        out_shape=jax.ShapeDtypeStruct((N, D), jnp.float32),
    )(acc, lin, dis, inv_deg, b, gamma, beta)


def _to_sc(lin):
    return lin.reshape(N, NW, FPT).transpose(1, 0, 2).reshape(NW, N * FPT)


def _from_sc(acc):
    return acc.reshape(NW, N, FPT).transpose(1, 0, 2).reshape(N, D)


def kernel(x, edge_index, W1, b1, gamma1, beta1, W2, b2, gamma2, beta2):
    row = edge_index[0].astype(jnp.int32)
    col = edge_index[1].astype(jnp.int32)

    parts = _hist_sc_kernel()(col)
    dis, inv_deg = _deg_finish(parts)

    lin1, lin1s = _matmul(x, W1, dis)
    acc1 = _from_sc(_agg_sc_kernel()(row, col, _to_sc(lin1s)))
    lin2, lin2s = _layer_mid(acc1, lin1, dis, inv_deg, b1, gamma1, beta1, W2)
    acc2 = _from_sc(_agg_sc_kernel()(row, col, _to_sc(lin2s)))
    return _layer_out(acc2, lin2, dis, inv_deg, b2, gamma2, beta2)


# speedup vs baseline: 17.2378x; 1.7055x over previous
import functools

import jax
import jax.numpy as jnp
from jax import lax
from jax.experimental import pallas as pl
from jax.experimental.pallas import tpu as pltpu
from jax.experimental.pallas import tpu_sc as plsc

N = 10000
E = 320000
D = 128

NC = 2
NS = 16
NW = NC * NS
FPT = D // NW
EPW = E // NW
EC = 8000
L = 16
UNROLL = 4


@functools.cache
def _mesh():
    return plsc.VectorSubcoreMesh(core_axis_name="c", subcore_axis_name="s",
                                  num_cores=NC, num_subcores=NS)


def _wid():
    return lax.axis_index("s") * NC + lax.axis_index("c")


def _zero_fill(ref, nwords):
    z = jnp.zeros((L,), jnp.float32)

    def body(i, _):
        ref[pl.ds(i * L, L)] = z
        return 0

    lax.fori_loop(0, nwords // L, body, 0)


@functools.cache
def _hist_sc_kernel():
    return pl.kernel(
        _hist_sc_body,
        out_type=jax.ShapeDtypeStruct((NW, N), jnp.float32),
        mesh=_mesh(),
        scratch_types=[
            pltpu.VMEM((EPW,), jnp.int32),
            pltpu.VMEM((N,), jnp.float32),
        ],
        compiler_params=pltpu.CompilerParams(needs_layout_passes=False),
    )


def _hist_sc_body(pk_hbm, out_hbm, pk_v, hist_v):
    w = _wid()
    pltpu.sync_copy(pk_hbm.at[pl.ds(w * EPW, EPW)], pk_v)
    _zero_fill(hist_v, N)
    ones = jnp.ones((L,), jnp.float32)

    @plsc.parallel_loop(0, EPW // L, unroll=UNROLL)
    def body(j):
        cols = pk_v[pl.ds(j * L, L)] & 0xFFFF
        plsc.addupdate_scatter(hist_v, [cols], ones)

    pltpu.sync_copy(hist_v, out_hbm.at[w])


@functools.cache
def _agg_sc_kernel():
    return pl.kernel(
        _agg_sc_body,
        out_type=jax.ShapeDtypeStruct((NW, N * FPT), jnp.float32),
        mesh=_mesh(),
        scratch_types=[
            pltpu.VMEM((N * FPT,), jnp.float32),
            pltpu.VMEM((N * FPT,), jnp.float32),
            pltpu.VMEM((EC,), jnp.int32),
        ],
        compiler_params=pltpu.CompilerParams(needs_layout_passes=False),
    )


def _agg_sc_body(pk_hbm, lin_hbm, out_hbm, lin_v, acc_v, pk_v):
    w = _wid()
    pltpu.sync_copy(lin_hbm.at[w], lin_v)
    _zero_fill(acc_v, N * FPT)

    def chunk(c, _):
        pltpu.sync_copy(pk_hbm.at[pl.ds(c * EC, EC)], pk_v)

        @plsc.parallel_loop(0, EC // L, unroll=UNROLL)
        def grp(j):
            pk = pk_v[pl.ds(j * L, L)]
            rows = pk >> 16
            cols = pk & 0xFFFF
            for f in range(FPT):
                v = plsc.load_gather(lin_v, [rows + (f * N)])
                plsc.addupdate_scatter(acc_v, [cols + (f * N)], v)

        return 0

    lax.fori_loop(0, E // EC, chunk, 0)
    pltpu.sync_copy(acc_v, out_hbm.at[w])


def _pack_body(row_ref, col_ref, pk_ref):
    pk_ref[...] = (row_ref[...] << 16) | col_ref[...]


def _pack(row, col):
    return pl.pallas_call(
        _pack_body,
        out_shape=jax.ShapeDtypeStruct((E,), jnp.int32),
    )(row, col)


def _deg_body(parts_ref, dis_ref, inv_ref):
    deg = jnp.sum(parts_ref[...], axis=0) + 1.0
    inv = 1.0 / deg
    inv_ref[...] = inv
    dis_ref[...] = jnp.sqrt(inv)


def _deg_finish(parts):
    return pl.pallas_call(
        _deg_body,
        out_shape=(
            jax.ShapeDtypeStruct((N,), jnp.float32),
            jax.ShapeDtypeStruct((N,), jnp.float32),
        ),
    )(parts)


def _mm_body(x_ref, w_ref, dis_ref, lin_ref, lins_ref):
    lin = jnp.dot(x_ref[...], w_ref[...], preferred_element_type=jnp.float32)
    lin_ref[...] = lin
    lins_ref[...] = lin * dis_ref[...].reshape(N, 1)


def _matmul(x, w, dis):
    return pl.pallas_call(
        _mm_body,
        out_shape=(
            jax.ShapeDtypeStruct((N, D), jnp.float32),
            jax.ShapeDtypeStruct((N, D), jnp.float32),
        ),
    )(x, w, dis)


def _bn_relu(t, gamma, beta):
    m = jnp.mean(t, axis=0)
    v = jnp.mean(t * t, axis=0) - m * m
    h = (t - m) * lax.rsqrt(v + 1e-5) * gamma + beta
    return jnp.maximum(h, 0.0)


def _mid_body(acc_ref, lin_ref, dis_ref, inv_ref, b_ref, g_ref, be_ref,
              w2_ref, lin2_ref, lin2s_ref):
    dis = dis_ref[...].reshape(N, 1)
    inv = inv_ref[...].reshape(N, 1)
    t = dis * acc_ref[...] + inv * lin_ref[...] + b_ref[...]
    h = _bn_relu(t, g_ref[...], be_ref[...])
    lin2 = jnp.dot(h, w2_ref[...], preferred_element_type=jnp.float32)
    lin2_ref[...] = lin2
    lin2s_ref[...] = lin2 * dis


def _layer_mid(acc, lin, dis, inv_deg, b, gamma, beta, w2):
    return pl.pallas_call(
        _mid_body,
        out_shape=(
            jax.ShapeDtypeStruct((N, D), jnp.float32),
            jax.ShapeDtypeStruct((N, D), jnp.float32),
        ),
    )(acc, lin, dis, inv_deg, b, gamma, beta, w2)


def _out_body(acc_ref, lin_ref, dis_ref, inv_ref, b_ref, g_ref, be_ref, o_ref):
    dis = dis_ref[...].reshape(N, 1)
    inv = inv_ref[...].reshape(N, 1)
    t = dis * acc_ref[...] + inv * lin_ref[...] + b_ref[...]
    o_ref[...] = _bn_relu(t, g_ref[...], be_ref[...])


def _layer_out(acc, lin, dis, inv_deg, b, gamma, beta):
    return pl.pallas_call(
        _out_body,
        out_shape=jax.ShapeDtypeStruct((N, D), jnp.float32),
    )(acc, lin, dis, inv_deg, b, gamma, beta)


def _to_sc(lin):
    return lin.reshape(N, NW, FPT).transpose(1, 2, 0).reshape(NW, N * FPT)


def _from_sc(acc):
    return acc.reshape(NW, FPT, N).transpose(2, 0, 1).reshape(N, D)


def kernel(x, edge_index, W1, b1, gamma1, beta1, W2, b2, gamma2, beta2):
    row = edge_index[0].astype(jnp.int32)
    col = edge_index[1].astype(jnp.int32)

    pk = _pack(row, col)
    parts = _hist_sc_kernel()(pk)
    dis, inv_deg = _deg_finish(parts)

    lin1, lin1s = _matmul(x, W1, dis)
    acc1 = _from_sc(_agg_sc_kernel()(pk, _to_sc(lin1s)))
    lin2, lin2s = _layer_mid(acc1, lin1, dis, inv_deg, b1, gamma1, beta1, W2)
    acc2 = _from_sc(_agg_sc_kernel()(pk, _to_sc(lin2s)))
    return _layer_out(acc2, lin2, dis, inv_deg, b2, gamma2, beta2)


# speedup vs baseline: 17.2476x; 1.0006x over previous
import functools

import jax
import jax.numpy as jnp
from jax import lax
from jax.experimental import pallas as pl
from jax.experimental.pallas import tpu as pltpu
from jax.experimental.pallas import tpu_sc as plsc

N = 10000
E = 320000
D = 128

NC = 2
NS = 16
NW = NC * NS
FPT = D // NW
EPW = E // NW
EC = 8000
L = 16
UNROLL = 4


@functools.cache
def _mesh():
    return plsc.VectorSubcoreMesh(core_axis_name="c", subcore_axis_name="s",
                                  num_cores=NC, num_subcores=NS)


def _wid():
    return lax.axis_index("s") * NC + lax.axis_index("c")


def _zero_fill(ref, nwords):
    z = jnp.zeros((L,), jnp.float32)

    def body(i, _):
        ref[pl.ds(i * L, L)] = z
        return 0

    lax.fori_loop(0, nwords // L, body, 0)


@functools.cache
def _hist_sc_kernel():
    return pl.kernel(
        _hist_sc_body,
        out_type=jax.ShapeDtypeStruct((NW, N), jnp.float32),
        mesh=_mesh(),
        scratch_types=[
            pltpu.VMEM((EPW,), jnp.int32),
            pltpu.VMEM((N,), jnp.float32),
        ],
        compiler_params=pltpu.CompilerParams(needs_layout_passes=False),
    )


def _hist_sc_body(pk_hbm, out_hbm, pk_v, hist_v):
    w = _wid()
    pltpu.sync_copy(pk_hbm.at[pl.ds(w * EPW, EPW)], pk_v)
    _zero_fill(hist_v, N)
    ones = jnp.ones((L,), jnp.float32)

    @plsc.parallel_loop(0, EPW // L, unroll=UNROLL)
    def body(j):
        cols = pk_v[pl.ds(j * L, L)] & 0xFFFF
        plsc.addupdate_scatter(hist_v, [cols], ones)

    pltpu.sync_copy(hist_v, out_hbm.at[w])


@functools.cache
def _agg_sc_kernel():
    return pl.kernel(
        _agg_sc_body,
        out_type=jax.ShapeDtypeStruct((NW, N * FPT), jnp.float32),
        mesh=_mesh(),
        scratch_types=[
            pltpu.VMEM((N * FPT,), jnp.float32),
            pltpu.VMEM((N * FPT,), jnp.float32),
            pltpu.VMEM((EC,), jnp.int32),
        ],
        compiler_params=pltpu.CompilerParams(needs_layout_passes=False),
    )


def _agg_sc_body(pk_hbm, lin_hbm, out_hbm, lin_v, acc_v, pk_v):
    w = _wid()
    pltpu.sync_copy(lin_hbm.at[w], lin_v)
    _zero_fill(acc_v, N * FPT)
    lin_f = [lin_v.at[pl.ds(f * N, N)] for f in range(FPT)]
    acc_f = [acc_v.at[pl.ds(f * N, N)] for f in range(FPT)]

    def chunk(c, _):
        pltpu.sync_copy(pk_hbm.at[pl.ds(c * EC, EC)], pk_v)

        @plsc.parallel_loop(0, EC // L, unroll=UNROLL)
        def grp(j):
            pk = pk_v[pl.ds(j * L, L)]
            rows = pk >> 16
            cols = pk & 0xFFFF
            for f in range(FPT):
                v = plsc.load_gather(lin_f[f], [rows])
                plsc.addupdate_scatter(acc_f[f], [cols], v)

        return 0

    lax.fori_loop(0, E // EC, chunk, 0)
    pltpu.sync_copy(acc_v, out_hbm.at[w])


def _pack_body(row_ref, col_ref, pk_ref):
    pk_ref[...] = (row_ref[...] << 16) | col_ref[...]


def _pack(row, col):
    return pl.pallas_call(
        _pack_body,
        out_shape=jax.ShapeDtypeStruct((E,), jnp.int32),
    )(row, col)


def _deg_body(parts_ref, dis_ref, inv_ref):
    deg = jnp.sum(parts_ref[...], axis=0) + 1.0
    inv = 1.0 / deg
    inv_ref[...] = inv
    dis_ref[...] = jnp.sqrt(inv)


def _deg_finish(parts):
    return pl.pallas_call(
        _deg_body,
        out_shape=(
            jax.ShapeDtypeStruct((N,), jnp.float32),
            jax.ShapeDtypeStruct((N,), jnp.float32),
        ),
    )(parts)


def _mm_body(x_ref, w_ref, dis_ref, lin_ref, lins_ref):
    lin = jnp.dot(x_ref[...], w_ref[...], preferred_element_type=jnp.float32)
    lin_ref[...] = lin
    lins_ref[...] = lin * dis_ref[...].reshape(N, 1)


def _matmul(x, w, dis):
    return pl.pallas_call(
        _mm_body,
        out_shape=(
            jax.ShapeDtypeStruct((N, D), jnp.float32),
            jax.ShapeDtypeStruct((N, D), jnp.float32),
        ),
    )(x, w, dis)


def _bn_relu(t, gamma, beta):
    m = jnp.mean(t, axis=0)
    v = jnp.mean(t * t, axis=0) - m * m
    h = (t - m) * lax.rsqrt(v + 1e-5) * gamma + beta
    return jnp.maximum(h, 0.0)


def _mid_body(acc_ref, lin_ref, dis_ref, inv_ref, b_ref, g_ref, be_ref,
              w2_ref, lin2_ref, lin2s_ref):
    dis = dis_ref[...].reshape(N, 1)
    inv = inv_ref[...].reshape(N, 1)
    t = dis * acc_ref[...] + inv * lin_ref[...] + b_ref[...]
    h = _bn_relu(t, g_ref[...], be_ref[...])
    lin2 = jnp.dot(h, w2_ref[...], preferred_element_type=jnp.float32)
    lin2_ref[...] = lin2
    lin2s_ref[...] = lin2 * dis


def _layer_mid(acc, lin, dis, inv_deg, b, gamma, beta, w2):
    return pl.pallas_call(
        _mid_body,
        out_shape=(
            jax.ShapeDtypeStruct((N, D), jnp.float32),
            jax.ShapeDtypeStruct((N, D), jnp.float32),
        ),
    )(acc, lin, dis, inv_deg, b, gamma, beta, w2)


def _out_body(acc_ref, lin_ref, dis_ref, inv_ref, b_ref, g_ref, be_ref, o_ref):
    dis = dis_ref[...].reshape(N, 1)
    inv = inv_ref[...].reshape(N, 1)
    t = dis * acc_ref[...] + inv * lin_ref[...] + b_ref[...]
    o_ref[...] = _bn_relu(t, g_ref[...], be_ref[...])


def _layer_out(acc, lin, dis, inv_deg, b, gamma, beta):
    return pl.pallas_call(
        _out_body,
        out_shape=jax.ShapeDtypeStruct((N, D), jnp.float32),
    )(acc, lin, dis, inv_deg, b, gamma, beta)


def _to_sc(lin):
    return lin.reshape(N, NW, FPT).transpose(1, 2, 0).reshape(NW, N * FPT)


def _from_sc(acc):
    return acc.reshape(NW, FPT, N).transpose(2, 0, 1).reshape(N, D)


def kernel(x, edge_index, W1, b1, gamma1, beta1, W2, b2, gamma2, beta2):
    row = edge_index[0].astype(jnp.int32)
    col = edge_index[1].astype(jnp.int32)

    pk = _pack(row, col)
    parts = _hist_sc_kernel()(pk)
    dis, inv_deg = _deg_finish(parts)

    lin1, lin1s = _matmul(x, W1, dis)
    acc1 = _from_sc(_agg_sc_kernel()(pk, _to_sc(lin1s)))
    lin2, lin2s = _layer_mid(acc1, lin1, dis, inv_deg, b1, gamma1, beta1, W2)
    acc2 = _from_sc(_agg_sc_kernel()(pk, _to_sc(lin2s)))
    return _layer_out(acc2, lin2, dis, inv_deg, b2, gamma2, beta2)


# speedup vs baseline: 19.1697x; 1.1114x over previous
import functools

import jax
import jax.numpy as jnp
from jax import lax
from jax.experimental import pallas as pl
from jax.experimental.pallas import tpu as pltpu
from jax.experimental.pallas import tpu_sc as plsc

N = 10000
E = 320000
D = 128

NC = 2
NS = 16
NW = NC * NS
FPT = D // NW
EPW = E // NW
EC = 40000
L = 16
UNROLL = 4


@functools.cache
def _mesh():
    return plsc.VectorSubcoreMesh(core_axis_name="c", subcore_axis_name="s",
                                  num_cores=NC, num_subcores=NS)


def _wid():
    return lax.axis_index("s") * NC + lax.axis_index("c")


def _zero_fill(ref, nwords):
    z = jnp.zeros((L,), jnp.float32)

    def body(i, _):
        ref[pl.ds(i * L, L)] = z
        return 0

    lax.fori_loop(0, nwords // L, body, 0)


@functools.cache
def _hist_sc_kernel():
    return pl.kernel(
        _hist_sc_body,
        out_type=jax.ShapeDtypeStruct((NW, N), jnp.float32),
        mesh=_mesh(),
        scratch_types=[
            pltpu.VMEM((EPW,), jnp.int32),
            pltpu.VMEM((N,), jnp.float32),
        ],
        compiler_params=pltpu.CompilerParams(needs_layout_passes=False),
    )


def _hist_sc_body(pk_hbm, out_hbm, pk_v, hist_v):
    w = _wid()
    pltpu.sync_copy(pk_hbm.at[pl.ds(w * EPW, EPW)], pk_v)
    _zero_fill(hist_v, N)
    ones = jnp.ones((L,), jnp.float32)

    @plsc.parallel_loop(0, EPW // L, unroll=UNROLL)
    def body(j):
        cols = pk_v[pl.ds(j * L, L)] & 0xFFFF
        plsc.addupdate_scatter(hist_v, [cols], ones)

    pltpu.sync_copy(hist_v, out_hbm.at[w])


@functools.cache
def _agg_sc_kernel():
    return pl.kernel(
        _agg_sc_body,
        out_type=jax.ShapeDtypeStruct((NW, N * FPT), jnp.float32),
        mesh=_mesh(),
        scratch_types=[
            pltpu.VMEM((N * FPT,), jnp.float32),
            pltpu.VMEM((N * FPT,), jnp.float32),
            pltpu.VMEM((EC,), jnp.int32),
        ],
        compiler_params=pltpu.CompilerParams(needs_layout_passes=False),
    )


def _agg_sc_body(pk_hbm, lin_hbm, out_hbm, lin_v, acc_v, pk_v):
    w = _wid()
    pltpu.sync_copy(lin_hbm.at[w], lin_v)
    _zero_fill(acc_v, N * FPT)
    lin_f = [lin_v.at[pl.ds(f * N, N)] for f in range(FPT)]
    acc_f = [acc_v.at[pl.ds(f * N, N)] for f in range(FPT)]

    def chunk(c, _):
        pltpu.sync_copy(pk_hbm.at[pl.ds(c * EC, EC)], pk_v)

        @plsc.parallel_loop(0, EC // L, unroll=UNROLL)
        def grp(j):
            pk = pk_v[pl.ds(j * L, L)]
            rows = pk >> 16
            cols = pk & 0xFFFF
            for f in range(FPT):
                v = plsc.load_gather(lin_f[f], [rows])
                plsc.addupdate_scatter(acc_f[f], [cols], v)

        return 0

    lax.fori_loop(0, E // EC, chunk, 0)
    pltpu.sync_copy(acc_v, out_hbm.at[w])


def _pack_body(row_ref, col_ref, pk_ref):
    pk_ref[...] = (row_ref[...] << 16) | col_ref[...]


def _pack(row, col):
    return pl.pallas_call(
        _pack_body,
        out_shape=jax.ShapeDtypeStruct((E,), jnp.int32),
    )(row, col)


def _deg_body(parts_ref, dis_ref, inv_ref):
    deg = jnp.sum(parts_ref[...], axis=0) + 1.0
    inv = 1.0 / deg
    inv_ref[...] = inv
    dis_ref[...] = jnp.sqrt(inv)


def _deg_finish(parts):
    return pl.pallas_call(
        _deg_body,
        out_shape=(
            jax.ShapeDtypeStruct((N,), jnp.float32),
            jax.ShapeDtypeStruct((N,), jnp.float32),
        ),
    )(parts)


def _mm_body(x_ref, w_ref, dis_ref, linT_ref, linsT_ref):
    linT = lax.dot_general(w_ref[...], x_ref[...], (((0,), (1,)), ((), ())),
                           preferred_element_type=jnp.float32)
    linT_ref[...] = linT
    linsT_ref[...] = linT * dis_ref[...].reshape(1, N)


def _matmul(x, w, dis):
    return pl.pallas_call(
        _mm_body,
        out_shape=(
            jax.ShapeDtypeStruct((D, N), jnp.float32),
            jax.ShapeDtypeStruct((D, N), jnp.float32),
        ),
    )(x, w, dis)


def _bn_relu_T(t, gamma, beta):
    m = jnp.mean(t, axis=1, keepdims=True)
    v = jnp.mean(t * t, axis=1, keepdims=True) - m * m
    h = (t - m) * lax.rsqrt(v + 1e-5) * gamma.reshape(D, 1) + beta.reshape(D, 1)
    return jnp.maximum(h, 0.0)


def _mid_body(accT_ref, linT_ref, dis_ref, inv_ref, b_ref, g_ref, be_ref,
              w2_ref, lin2T_ref, lin2sT_ref):
    dis = dis_ref[...].reshape(1, N)
    inv = inv_ref[...].reshape(1, N)
    t = dis * accT_ref[...] + inv * linT_ref[...] + b_ref[...].reshape(D, 1)
    h = _bn_relu_T(t, g_ref[...], be_ref[...])
    lin2T = lax.dot_general(w2_ref[...], h, (((0,), (0,)), ((), ())),
                            preferred_element_type=jnp.float32)
    lin2T_ref[...] = lin2T
    lin2sT_ref[...] = lin2T * dis


def _layer_mid(accT, linT, dis, inv_deg, b, gamma, beta, w2):
    return pl.pallas_call(
        _mid_body,
        out_shape=(
            jax.ShapeDtypeStruct((D, N), jnp.float32),
            jax.ShapeDtypeStruct((D, N), jnp.float32),
        ),
    )(accT, linT, dis, inv_deg, b, gamma, beta, w2)


def _out_body(accT_ref, linT_ref, dis_ref, inv_ref, b_ref, g_ref, be_ref, o_ref):
    dis = dis_ref[...].reshape(1, N)
    inv = inv_ref[...].reshape(1, N)
    t = dis * accT_ref[...] + inv * linT_ref[...] + b_ref[...].reshape(D, 1)
    h = _bn_relu_T(t, g_ref[...], be_ref[...])
    o_ref[...] = h.T


def _layer_out(accT, linT, dis, inv_deg, b, gamma, beta):
    return pl.pallas_call(
        _out_body,
        out_shape=jax.ShapeDtypeStruct((N, D), jnp.float32),
    )(accT, linT, dis, inv_deg, b, gamma, beta)


def kernel(x, edge_index, W1, b1, gamma1, beta1, W2, b2, gamma2, beta2):
    row = edge_index[0].astype(jnp.int32)
    col = edge_index[1].astype(jnp.int32)

    pk = _pack(row, col)
    parts = _hist_sc_kernel()(pk)
    dis, inv_deg = _deg_finish(parts)

    lin1T, lin1sT = _matmul(x, W1, dis)
    acc1T = _agg_sc_kernel()(pk, lin1sT.reshape(NW, N * FPT)).reshape(D, N)
    lin2T, lin2sT = _layer_mid(acc1T, lin1T, dis, inv_deg, b1, gamma1, beta1, W2)
    acc2T = _agg_sc_kernel()(pk, lin2sT.reshape(NW, N * FPT)).reshape(D, N)
    return _layer_out(acc2T, lin2T, dis, inv_deg, b2, gamma2, beta2)


# speedup vs baseline: 21.7081x; 1.1324x over previous
import functools

import jax
import jax.numpy as jnp
from jax import lax
from jax.experimental import pallas as pl
from jax.experimental.pallas import tpu as pltpu
from jax.experimental.pallas import tpu_sc as plsc

N = 10000
E = 320000
D = 128

NC = 2
NS = 16
NW = NC * NS
FPT = D // NW
PPT = FPT // 2
EPW = E // NW
EC = 40000
L = 16
UNROLL = 4


@functools.cache
def _mesh():
    return plsc.VectorSubcoreMesh(core_axis_name="c", subcore_axis_name="s",
                                  num_cores=NC, num_subcores=NS)


def _wid():
    return lax.axis_index("s") * NC + lax.axis_index("c")


def _zero_fill(ref, nwords):
    z = jnp.zeros((L,), jnp.float32)

    def body(i, _):
        ref[pl.ds(i * L, L)] = z
        return 0

    lax.fori_loop(0, nwords // L, body, 0)


@functools.cache
def _hist_sc_kernel():
    return pl.kernel(
        _hist_sc_body,
        out_type=jax.ShapeDtypeStruct((NW, N), jnp.float32),
        mesh=_mesh(),
        scratch_types=[
            pltpu.VMEM((EPW,), jnp.int32),
            pltpu.VMEM((N,), jnp.float32),
        ],
        compiler_params=pltpu.CompilerParams(needs_layout_passes=False),
    )


def _hist_sc_body(pk_hbm, out_hbm, pk_v, hist_v):
    w = _wid()
    pltpu.sync_copy(pk_hbm.at[pl.ds(w * EPW, EPW)], pk_v)
    _zero_fill(hist_v, N)
    ones = jnp.ones((L,), jnp.float32)

    @plsc.parallel_loop(0, EPW // L, unroll=UNROLL)
    def body(j):
        cols = pk_v[pl.ds(j * L, L)] & 0xFFFF
        plsc.addupdate_scatter(hist_v, [cols], ones)

    pltpu.sync_copy(hist_v, out_hbm.at[w])


@functools.cache
def _agg_sc_kernel():
    return pl.kernel(
        _agg_sc_body,
        out_type=jax.ShapeDtypeStruct((NW, N * FPT), jnp.float32),
        mesh=_mesh(),
        scratch_types=[
            pltpu.VMEM((N * PPT,), jnp.int32),
            pltpu.VMEM((N * FPT,), jnp.float32),
            pltpu.VMEM((EC,), jnp.int32),
        ],
        compiler_params=pltpu.CompilerParams(needs_layout_passes=False),
    )


def _agg_sc_body(pk_hbm, lin_hbm, out_hbm, lin_v, acc_v, pk_v):
    w = _wid()
    pltpu.sync_copy(lin_hbm.at[w], lin_v)
    _zero_fill(acc_v, N * FPT)
    lin_p = [lin_v.at[pl.ds(p * N, N)] for p in range(PPT)]
    acc_f = [acc_v.at[pl.ds(f * N, N)] for f in range(FPT)]
    hi_mask = jnp.full((L,), -65536, jnp.int32)

    def chunk(c, _):
        pltpu.sync_copy(pk_hbm.at[pl.ds(c * EC, EC)], pk_v)

        @plsc.parallel_loop(0, EC // L, unroll=UNROLL)
        def grp(j):
            pk = pk_v[pl.ds(j * L, L)]
            rows = pk >> 16
            cols = pk & 0xFFFF
            for p in range(PPT):
                pair = plsc.load_gather(lin_p[p], [rows])
                lo = plsc.bitcast(pair << 16, jnp.float32)
                hi = plsc.bitcast(pair & hi_mask, jnp.float32)
                plsc.addupdate_scatter(acc_f[2 * p], [cols], lo)
                plsc.addupdate_scatter(acc_f[2 * p + 1], [cols], hi)

        return 0

    lax.fori_loop(0, E // EC, chunk, 0)
    pltpu.sync_copy(acc_v, out_hbm.at[w])


def _pack_body(row_ref, col_ref, pk_ref):
    pk_ref[...] = (row_ref[...] << 16) | col_ref[...]


def _pack(row, col):
    return pl.pallas_call(
        _pack_body,
        out_shape=jax.ShapeDtypeStruct((E,), jnp.int32),
    )(row, col)


def _deg_body(parts_ref, dis_ref, inv_ref):
    deg = jnp.sum(parts_ref[...], axis=0) + 1.0
    inv = 1.0 / deg
    inv_ref[...] = inv
    dis_ref[...] = jnp.sqrt(inv)


def _deg_finish(parts):
    return pl.pallas_call(
        _deg_body,
        out_shape=(
            jax.ShapeDtypeStruct((N,), jnp.float32),
            jax.ShapeDtypeStruct((N,), jnp.float32),
        ),
    )(parts)


def _pack_bf16_pairs(even, odd):
    a = lax.bitcast_convert_type(even.astype(jnp.bfloat16),
                                 jnp.uint16).astype(jnp.uint32)
    b = lax.bitcast_convert_type(odd.astype(jnp.bfloat16),
                                 jnp.uint16).astype(jnp.uint32)
    return lax.bitcast_convert_type((b << 16) | a, jnp.int32)


def _mm_body(x_ref, w_ref, we_ref, wo_ref, dis_ref, linT_ref, linsP_ref):
    dis = dis_ref[...].reshape(1, N)
    linT_ref[...] = lax.dot_general(
        w_ref[...], x_ref[...], (((0,), (1,)), ((), ())),
        preferred_element_type=jnp.float32)
    even = lax.dot_general(we_ref[...], x_ref[...], (((0,), (1,)), ((), ())),
                           preferred_element_type=jnp.float32) * dis
    odd = lax.dot_general(wo_ref[...], x_ref[...], (((0,), (1,)), ((), ())),
                          preferred_element_type=jnp.float32) * dis
    linsP_ref[...] = _pack_bf16_pairs(even, odd)


def _matmul(x, w, we, wo, dis):
    return pl.pallas_call(
        _mm_body,
        out_shape=(
            jax.ShapeDtypeStruct((D, N), jnp.float32),
            jax.ShapeDtypeStruct((D // 2, N), jnp.int32),
        ),
    )(x, w, we, wo, dis)


def _bn_relu_T(t, gamma, beta):
    m = jnp.mean(t, axis=1, keepdims=True)
    v = jnp.mean(t * t, axis=1, keepdims=True) - m * m
    h = (t - m) * lax.rsqrt(v + 1e-5) * gamma.reshape(D, 1) + beta.reshape(D, 1)
    return jnp.maximum(h, 0.0)


def _mid_body(accT_ref, linT_ref, dis_ref, inv_ref, b_ref, g_ref, be_ref,
              w2_ref, w2e_ref, w2o_ref, lin2T_ref, lin2sP_ref):
    dis = dis_ref[...].reshape(1, N)
    inv = inv_ref[...].reshape(1, N)
    t = dis * accT_ref[...] + inv * linT_ref[...] + b_ref[...].reshape(D, 1)
    h = _bn_relu_T(t, g_ref[...], be_ref[...])
    lin2T_ref[...] = lax.dot_general(w2_ref[...], h, (((0,), (0,)), ((), ())),
                                     preferred_element_type=jnp.float32)
    even = lax.dot_general(w2e_ref[...], h, (((0,), (0,)), ((), ())),
                           preferred_element_type=jnp.float32) * dis
    odd = lax.dot_general(w2o_ref[...], h, (((0,), (0,)), ((), ())),
                          preferred_element_type=jnp.float32) * dis
    lin2sP_ref[...] = _pack_bf16_pairs(even, odd)


def _layer_mid(accT, linT, dis, inv_deg, b, gamma, beta, w2, w2e, w2o):
    return pl.pallas_call(
        _mid_body,
        out_shape=(
            jax.ShapeDtypeStruct((D, N), jnp.float32),
            jax.ShapeDtypeStruct((D // 2, N), jnp.int32),
        ),
    )(accT, linT, dis, inv_deg, b, gamma, beta, w2, w2e, w2o)


def _out_body(accT_ref, linT_ref, dis_ref, inv_ref, b_ref, g_ref, be_ref, o_ref):
    dis = dis_ref[...].reshape(1, N)
    inv = inv_ref[...].reshape(1, N)
    t = dis * accT_ref[...] + inv * linT_ref[...] + b_ref[...].reshape(D, 1)
    h = _bn_relu_T(t, g_ref[...], be_ref[...])
    o_ref[...] = h.T


def _layer_out(accT, linT, dis, inv_deg, b, gamma, beta):
    return pl.pallas_call(
        _out_body,
        out_shape=jax.ShapeDtypeStruct((N, D), jnp.float32),
    )(accT, linT, dis, inv_deg, b, gamma, beta)


def kernel(x, edge_index, W1, b1, gamma1, beta1, W2, b2, gamma2, beta2):
    row = edge_index[0].astype(jnp.int32)
    col = edge_index[1].astype(jnp.int32)

    pk = _pack(row, col)
    parts = _hist_sc_kernel()(pk)
    dis, inv_deg = _deg_finish(parts)

    lin1T, lin1sP = _matmul(x, W1, W1[:, 0::2], W1[:, 1::2], dis)
    acc1T = _agg_sc_kernel()(pk, lin1sP.reshape(NW, N * PPT)).reshape(D, N)
    lin2T, lin2sP = _layer_mid(acc1T, lin1T, dis, inv_deg, b1, gamma1, beta1,
                               W2, W2[:, 0::2], W2[:, 1::2])
    acc2T = _agg_sc_kernel()(pk, lin2sP.reshape(NW, N * PPT)).reshape(D, N)
    return _layer_out(acc2T, lin2T, dis, inv_deg, b2, gamma2, beta2)
